# baseline (speedup 1.0000x reference)
import functools

import jax
import jax.numpy as jnp
from jax import lax
from jax.experimental import pallas as pl
from jax.experimental.pallas import tpu as pltpu
from jax.experimental.pallas import tpu_sc as plsc

_NC = 2
_NS = 16
_IDX_CHUNK = 128


def _node_body(x_ref, wnnT_ref, bnn_ref, root_ref, u_ref, xr_ref):
    xb = x_ref[...]
    u_ref[...] = (
        jnp.dot(xb, wnnT_ref[...], preferred_element_type=jnp.float32)
        + bnn_ref[...]
    )
    xr_ref[...] = jnp.dot(xb, root_ref[...], preferred_element_type=jnp.float32)


def _node_transform(x, wnnT, bnn2, root):
    n, d = x.shape
    bn = 2000
    grid = n // bn
    return pl.pallas_call(
        _node_body,
        grid=(grid,),
        in_specs=[
            pl.BlockSpec((bn, d), lambda i: (i, 0)),
            pl.BlockSpec((d, d), lambda i: (0, 0)),
            pl.BlockSpec((1, d), lambda i: (0, 0)),
            pl.BlockSpec((d, d), lambda i: (0, 0)),
        ],
        out_specs=[
            pl.BlockSpec((bn, d), lambda i: (i, 0)),
            pl.BlockSpec((bn, d), lambda i: (i, 0)),
        ],
        out_shape=[
            jax.ShapeDtypeStruct((n, d), jnp.float32),
            jax.ShapeDtypeStruct((n, d), jnp.float32),
        ],
    )(x, wnnT, bnn2, root)


def _sc_gather(u, src2d, dst2d, e, d):
    mesh = plsc.VectorSubcoreMesh(core_axis_name="c", subcore_axis_name="s")
    bpw = e // (_NC * _NS)
    nch = bpw // _IDX_CHUNK

    @functools.partial(
        pl.kernel,
        mesh=mesh,
        out_type=[
            jax.ShapeDtypeStruct((e, d), jnp.float32),
            jax.ShapeDtypeStruct((e, d), jnp.float32),
        ],
        scratch_types=[
            pltpu.VMEM((nch, _IDX_CHUNK), jnp.int32),
            pltpu.VMEM((bpw, d), jnp.float32),
            pltpu.SemaphoreType.DMA,
        ],
    )
    def k(u_hbm, src_hbm, dst_hbm, wj_hbm, wi_hbm, idx_v, rows_v, sem):
        wid = lax.axis_index("s") * _NC + lax.axis_index("c")
        base = wid * bpw
        rowbase = wid * nch
        pltpu.sync_copy(src_hbm.at[pl.ds(rowbase, nch)], idx_v)
        for j in range(nch):
            pltpu.async_copy(
                u_hbm.at[idx_v.at[j]],
                rows_v.at[pl.ds(j * _IDX_CHUNK, _IDX_CHUNK)],
                sem,
            ).wait()
        pltpu.sync_copy(rows_v, wj_hbm.at[pl.ds(base, bpw)])
        pltpu.sync_copy(dst_hbm.at[pl.ds(rowbase, nch)], idx_v)
        for j in range(nch):
            pltpu.async_copy(
                u_hbm.at[idx_v.at[j]],
                rows_v.at[pl.ds(j * _IDX_CHUNK, _IDX_CHUNK)],
                sem,
            ).wait()
        pltpu.sync_copy(rows_v, wi_hbm.at[pl.ds(base, bpw)])

    return k(u, src2d, dst2d)


def _edge_body(wi_ref, wj_ref, ea_ref, wnnT_ref, bnn_ref, a1_ref, a2_ref,
               a3_ref, ba1_ref, wa2_ref, ex_ref):
    we = (
        jnp.dot(ea_ref[...], wnnT_ref[...], preferred_element_type=jnp.float32)
        + bnn_ref[...]
    )
    pre = (
        jnp.dot(wi_ref[...], a1_ref[...], preferred_element_type=jnp.float32)
        + jnp.dot(wj_ref[...], a2_ref[...], preferred_element_type=jnp.float32)
        + jnp.dot(we, a3_ref[...], preferred_element_type=jnp.float32)
        + ba1_ref[...]
    )
    h = jnp.tanh(pre)
    att = jnp.sum(h * wa2_ref[...], axis=1)
    att = jnp.where(att >= 0, att, 0.2 * att)
    ex_ref[...] = jnp.exp(att)[None, :]


def _edge_scores(wi, wj, edge_attr, wnnT, bnn2, a1T, a2T, a3T, ba12, wa2row):
    e, d = wi.shape
    be = 1024
    grid = e // be
    wspec = pl.BlockSpec((d, d), lambda i: (0, 0))
    rspec = pl.BlockSpec((1, d), lambda i: (0, 0))
    return pl.pallas_call(
        _edge_body,
        grid=(grid,),
        in_specs=[
            pl.BlockSpec((be, d), lambda i: (i, 0)),
            pl.BlockSpec((be, d), lambda i: (i, 0)),
            pl.BlockSpec((be, d), lambda i: (i, 0)),
            wspec, rspec, wspec, wspec, wspec, rspec, rspec,
        ],
        out_specs=pl.BlockSpec((1, be), lambda i: (0, i)),
        out_shape=jax.ShapeDtypeStruct((1, e), jnp.float32),
    )(wi, wj, edge_attr, wnnT, bnn2, a1T, a2T, a3T, ba12, wa2row)


def _mask_body(mask_hbm, ex_ref, wj_ref, msg_ref, mb0, mb1, sem0, sem1,
               *, br, e, ck):
    m8 = mask_hbm
    nblk = e // br
    bufs = (mb0, mb1)
    sems = (sem0, sem1)

    def dma(r):
        b = r % 2
        return pltpu.make_async_copy(
            m8.at[pl.ds(r * br, br), :], bufs[b], sems[b]
        )

    dma(0).start()
    for r in range(nblk):
        if r + 1 < nblk:
            dma(r + 1).start()
        dma(r).wait()
        mref = bufs[r % 2]
        acc = jnp.zeros((br, ck), jnp.float32)
        for k in range(e // ck):
            mf = mref[:, k * ck:(k + 1) * ck].astype(jnp.float32)
            exc = ex_ref[0, k * ck:(k + 1) * ck]
            acc = acc + mf * exc[None, :]
        den = jnp.sum(acc, axis=1)
        dmf = mref[:, r * br:(r + 1) * br].astype(jnp.float32)
        row_io = lax.broadcasted_iota(jnp.int32, (br, br), 0)
        col_io = lax.broadcasted_iota(jnp.int32, (br, br), 1)
        ex_rows = ex_ref[0, r * br:(r + 1) * br]
        num = jnp.sum(
            jnp.where(row_io == col_io, dmf * ex_rows[None, :], 0.0), axis=1
        )
        s = jnp.where(den > 0, num / den, 1.0 / e)
        msg_ref[r * br:(r + 1) * br, :] = (
            wj_ref[r * br:(r + 1) * br, :] * s[:, None]
        )


def _mask_messages(edge_masks, ex, wj):
    e, d = wj.shape
    br = 512
    ck = 1024
    body = functools.partial(_mask_body, br=br, e=e, ck=ck)
    return pl.pallas_call(
        body,
        in_specs=[
            pl.BlockSpec(memory_space=pl.ANY),
            pl.BlockSpec((1, e), lambda: (0, 0)),
            pl.BlockSpec((e, d), lambda: (0, 0)),
        ],
        out_specs=pl.BlockSpec((e, d), lambda: (0, 0)),
        out_shape=jax.ShapeDtypeStruct((e, d), jnp.float32),
        scratch_shapes=[
            pltpu.VMEM((br, e), jnp.int8),
            pltpu.VMEM((br, e), jnp.int8),
            pltpu.SemaphoreType.DMA,
            pltpu.SemaphoreType.DMA,
        ],
    )(edge_masks, ex, wj)


def _sc_scatter(msg, dst2d, zrows, n_pad, d, e):
    mesh = plsc.VectorSubcoreMesh(core_axis_name="c", subcore_axis_name="s")
    epc = e // _NC
    eps = epc // _NS
    nch = eps // _IDX_CHUNK
    rows_pc = n_pad // _NS

    @functools.partial(
        pl.kernel,
        mesh=mesh,
        out_type=jax.ShapeDtypeStruct((_NC, n_pad, d), jnp.float32),
        scratch_types=[
            pltpu.VMEM((nch, _IDX_CHUNK), jnp.int32),
            pltpu.VMEM((eps, d), jnp.float32),
            pltpu.VMEM_SHARED((n_pad, d), jnp.float32),
            pltpu.SemaphoreType.DMA,
        ],
    )
    def k(msg_hbm, dst_hbm, z_hbm, out_hbm, idx_v, rows_v, acc_sh, sem):
        cid = lax.axis_index("c")
        sid = lax.axis_index("s")
        base = cid * epc + sid * eps
        rowbase = cid * (epc // _IDX_CHUNK) + sid * nch
        pltpu.sync_copy(dst_hbm.at[pl.ds(rowbase, nch)], idx_v)
        pltpu.sync_copy(msg_hbm.at[pl.ds(base, eps)], rows_v)
        pltpu.sync_copy(
            z_hbm.at[pl.ds(sid * rows_pc, rows_pc)],
            acc_sh.at[pl.ds(sid * rows_pc, rows_pc)],
        )
        plsc.subcore_barrier()
        for j in range(nch):
            pltpu.sync_copy(
                rows_v.at[pl.ds(j * _IDX_CHUNK, _IDX_CHUNK)],
                acc_sh.at[idx_v.at[j]],
                add=True,
            )
        plsc.subcore_barrier()
        pltpu.sync_copy(
            acc_sh.at[pl.ds(sid * rows_pc, rows_pc)],
            out_hbm.at[cid].at[pl.ds(sid * rows_pc, rows_pc)],
        )

    return k(msg, dst2d, zrows)


def _final_body(agg_ref, xr_ref, bias_ref, g1_ref, be1_ref, g2_ref, be2_ref,
                wlinT_ref, blin_ref, out_ref):
    o = agg_ref[0] + agg_ref[1] + xr_ref[...] + bias_ref[...]
    mu = jnp.mean(o, axis=1, keepdims=True)
    var = jnp.mean((o - mu) * (o - mu), axis=1, keepdims=True)
    h1 = (o - mu) * lax.rsqrt(var + 1e-5) * g1_ref[...] + be1_ref[...]
    t = (
        h1
        + jnp.dot(h1, wlinT_ref[...], preferred_element_type=jnp.float32)
        + blin_ref[...]
    )
    mu2 = jnp.mean(t, axis=1, keepdims=True)
    var2 = jnp.mean((t - mu2) * (t - mu2), axis=1, keepdims=True)
    out_ref[...] = (t - mu2) * lax.rsqrt(var2 + 1e-5) * g2_ref[...] + be2_ref[...]


def _finalize(agg2, xr, bias2, g12, be12, g22, be22, wlinT, blin2):
    n, d = xr.shape
    bn = 2000
    grid = n // bn
    rspec = pl.BlockSpec((1, d), lambda i: (0, 0))
    return pl.pallas_call(
        _final_body,
        grid=(grid,),
        in_specs=[
            pl.BlockSpec((_NC, bn, d), lambda i: (0, i, 0)),
            pl.BlockSpec((bn, d), lambda i: (i, 0)),
            rspec, rspec, rspec, rspec, rspec,
            pl.BlockSpec((d, d), lambda i: (0, 0)),
            rspec,
        ],
        out_specs=pl.BlockSpec((bn, d), lambda i: (i, 0)),
        out_shape=jax.ShapeDtypeStruct((n, d), jnp.float32),
    )(agg2, xr, bias2, g12, be12, g22, be22, wlinT, blin2)


def kernel(x, edge_index, edge_attr, edge_masks, W_nn, b_nn, W_a1, b_a1,
           W_a2, root, bias, g1, be1, g2, be2, W_lin, b_lin):
    n, d = x.shape
    e = edge_index.shape[1]

    src2d = edge_index[0].astype(jnp.int32).reshape(e // _IDX_CHUNK, _IDX_CHUNK)
    dst2d = edge_index[1].astype(jnp.int32).reshape(e // _IDX_CHUNK, _IDX_CHUNK)

    wnnT = W_nn.T
    bnn2 = b_nn.reshape(1, d)
    a1T = W_a1[:, :d].T
    a2T = W_a1[:, d:2 * d].T
    a3T = W_a1[:, 2 * d:].T
    ba12 = b_a1.reshape(1, d)
    wa2row = W_a2.reshape(1, d)
    wlinT = W_lin.T
    n_pad = ((n + _NS * 8 - 1) // (_NS * 8)) * (_NS * 8)
    zrows = jnp.zeros((n_pad, d), jnp.float32)

    u, xr = _node_transform(x, wnnT, bnn2, root)
    wj, wi = _sc_gather(u, src2d, dst2d, e, d)
    ex = _edge_scores(wi, wj, edge_attr, wnnT, bnn2, a1T, a2T, a3T, ba12,
                      wa2row)
    msg = _mask_messages(edge_masks.view(jnp.int8), ex, wj)
    agg2 = _sc_scatter(msg, dst2d, zrows, n_pad, d, e)
    return _finalize(agg2, xr, bias.reshape(1, d), g1.reshape(1, d),
                     be1.reshape(1, d), g2.reshape(1, d), be2.reshape(1, d),
                     wlinT, b_lin.reshape(1, d))


# speedup vs baseline: 1.1243x; 1.1243x over previous
import functools

import jax
import jax.numpy as jnp
from jax import lax
from jax.experimental import pallas as pl
from jax.experimental.pallas import tpu as pltpu
from jax.experimental.pallas import tpu_sc as plsc

_NC = 2
_NS = 16
_IDX_CHUNK = 128


def _node_body(x_ref, wnnT_ref, bnn_ref, u_ref, z_ref):
    u_ref[...] = (
        jnp.dot(x_ref[...], wnnT_ref[...], preferred_element_type=jnp.float32)
        + bnn_ref[...]
    )
    z_ref[...] = jnp.zeros_like(z_ref)


def _node_transform(x, wnnT, bnn2, n_pad):
    n, d = x.shape
    grid = 5
    bn = n // grid
    bz = n_pad // grid
    return pl.pallas_call(
        _node_body,
        grid=(grid,),
        in_specs=[
            pl.BlockSpec((bn, d), lambda i: (i, 0)),
            pl.BlockSpec((d, d), lambda i: (0, 0)),
            pl.BlockSpec((1, d), lambda i: (0, 0)),
        ],
        out_specs=[
            pl.BlockSpec((bn, d), lambda i: (i, 0)),
            pl.BlockSpec((bz, d), lambda i: (i, 0)),
        ],
        out_shape=[
            jax.ShapeDtypeStruct((n, d), jnp.float32),
            jax.ShapeDtypeStruct((n_pad, d), jnp.float32),
        ],
    )(x, wnnT, bnn2)


def _sc_gather(u, src2d, dst2d, e, d):
    mesh = plsc.VectorSubcoreMesh(core_axis_name="c", subcore_axis_name="s")
    epc = e // _NC
    bpw = epc // _NS
    nch = bpw // _IDX_CHUNK

    @functools.partial(
        pl.kernel,
        mesh=mesh,
        out_type=[
            jax.ShapeDtypeStruct((_NC, epc, d), jnp.float32),
            jax.ShapeDtypeStruct((_NC, epc, d), jnp.float32),
        ],
        scratch_types=[
            pltpu.VMEM((nch, _IDX_CHUNK), jnp.int32),
            pltpu.VMEM((nch, _IDX_CHUNK), jnp.int32),
            pltpu.VMEM((bpw, d), jnp.float32),
            pltpu.VMEM((bpw, d), jnp.float32),
            pltpu.SemaphoreType.DMA,
            pltpu.SemaphoreType.DMA,
        ],
    )
    def k(u_hbm, src_hbm, dst_hbm, wj_hbm, wi_hbm, sidx, didx, srows, drows,
          sem_s, sem_d):
        cid = lax.axis_index("c")
        sid = lax.axis_index("s")
        rowbase = cid * (epc // _IDX_CHUNK) + sid * nch
        pltpu.sync_copy(src_hbm.at[pl.ds(rowbase, nch)], sidx)
        pltpu.sync_copy(dst_hbm.at[pl.ds(rowbase, nch)], didx)
        cps = [
            pltpu.async_copy(
                u_hbm.at[sidx.at[j]],
                srows.at[pl.ds(j * _IDX_CHUNK, _IDX_CHUNK)],
                sem_s,
            )
            for j in range(nch)
        ]
        cpd = [
            pltpu.async_copy(
                u_hbm.at[didx.at[j]],
                drows.at[pl.ds(j * _IDX_CHUNK, _IDX_CHUNK)],
                sem_d,
            )
            for j in range(nch)
        ]
        for c in cps:
            c.wait()
        pltpu.sync_copy(srows, wj_hbm.at[cid].at[pl.ds(sid * bpw, bpw)])
        for c in cpd:
            c.wait()
        pltpu.sync_copy(drows, wi_hbm.at[cid].at[pl.ds(sid * bpw, bpw)])

    return k(u, src2d, dst2d)


def _edge_mask_body(wi_ref, wj_ref, ea_ref, wnnT_ref, bnn_ref, a1_ref,
                    a2_ref, a3_ref, ba1_ref, wa2_ref, mask_hbm, msg_ref,
                    mb0, mb1, mb2, mb3, s0, s1, s2, s3, *, br, e, ck, d):
    bufs = (mb0, mb1, mb2, mb3)
    sems = (s0, s1, s2, s3)
    nblk = e // br

    def dma(r):
        b = r % 4
        return pltpu.make_async_copy(
            mask_hbm.at[pl.ds(r * br, br), :], bufs[b], sems[b]
        )

    for r in range(3):
        dma(r).start()

    wiv = wi_ref[...].reshape(e, d)
    wjv = wj_ref[...].reshape(e, d)
    we = (
        jnp.dot(ea_ref[...], wnnT_ref[...], preferred_element_type=jnp.float32)
        + bnn_ref[...]
    )
    pre = (
        jnp.dot(wiv, a1_ref[...], preferred_element_type=jnp.float32)
        + jnp.dot(wjv, a2_ref[...], preferred_element_type=jnp.float32)
        + jnp.dot(we, a3_ref[...], preferred_element_type=jnp.float32)
        + ba1_ref[...]
    )
    h = jnp.tanh(pre)
    att = jnp.sum(h * wa2_ref[...], axis=1)
    att = jnp.where(att >= 0, att, 0.2 * att)
    ex = jnp.exp(att)
    exi = lax.bitcast_convert_type(ex, jnp.int32)

    for r in range(nblk):
        if r + 3 < nblk:
            dma(r + 3).start()
        dma(r).wait()
        mref = bufs[r % 4]
        acc = jnp.zeros((br, ck), jnp.float32)
        for k in range(e // ck):
            mi = mref[:, k * ck:(k + 1) * ck].astype(jnp.int32)
            prod = mi * exi[k * ck:(k + 1) * ck][None, :]
            acc = acc + lax.bitcast_convert_type(prod, jnp.float32)
        den = jnp.sum(acc, axis=1)
        dmf = mref[:, r * br:(r + 1) * br].astype(jnp.float32)
        row_io = lax.broadcasted_iota(jnp.int32, (br, br), 0)
        col_io = lax.broadcasted_iota(jnp.int32, (br, br), 1)
        ex_rows = ex[r * br:(r + 1) * br]
        num = jnp.sum(
            jnp.where(row_io == col_io, dmf * ex_rows[None, :], 0.0), axis=1
        )
        s = jnp.where(den > 0, num / den, 1.0 / e)
        msg_ref[r * br:(r + 1) * br, :] = (
            wjv[r * br:(r + 1) * br, :] * s[:, None]
        )


def _edge_mask_messages(wi2, wj2, edge_attr, edge_masks, wnnT, bnn2, a1T,
                        a2T, a3T, ba12, wa2row):
    e, d = edge_attr.shape
    br = 512
    ck = 1024
    body = functools.partial(_edge_mask_body, br=br, e=e, ck=ck, d=d)
    wspec = pl.BlockSpec((d, d), lambda: (0, 0))
    rspec = pl.BlockSpec((1, d), lambda: (0, 0))
    full2 = pl.BlockSpec((_NC, e // _NC, d), lambda: (0, 0, 0))
    return pl.pallas_call(
        body,
        in_specs=[
            full2,
            full2,
            pl.BlockSpec((e, d), lambda: (0, 0)),
            wspec, rspec, wspec, wspec, wspec, rspec, rspec,
            pl.BlockSpec(memory_space=pl.ANY),
        ],
        out_specs=pl.BlockSpec((e, d), lambda: (0, 0)),
        out_shape=jax.ShapeDtypeStruct((e, d), jnp.float32),
        scratch_shapes=[
            pltpu.VMEM((br, e), jnp.int8),
            pltpu.VMEM((br, e), jnp.int8),
            pltpu.VMEM((br, e), jnp.int8),
            pltpu.VMEM((br, e), jnp.int8),
            pltpu.SemaphoreType.DMA,
            pltpu.SemaphoreType.DMA,
            pltpu.SemaphoreType.DMA,
            pltpu.SemaphoreType.DMA,
        ],
    )(wi2, wj2, edge_attr, wnnT, bnn2, a1T, a2T, a3T, ba12, wa2row,
      edge_masks)


def _sc_scatter(msg, dst2d, zrows, n_pad, d, e):
    mesh = plsc.VectorSubcoreMesh(core_axis_name="c", subcore_axis_name="s")
    epc = e // _NC
    eps = epc // _NS
    nch = eps // _IDX_CHUNK
    rows_pc = n_pad // _NS

    @functools.partial(
        pl.kernel,
        mesh=mesh,
        out_type=jax.ShapeDtypeStruct((_NC, n_pad, d), jnp.float32),
        scratch_types=[
            pltpu.VMEM((nch, _IDX_CHUNK), jnp.int32),
            pltpu.VMEM((eps, d), jnp.float32),
            pltpu.VMEM_SHARED((n_pad, d), jnp.float32),
            pltpu.SemaphoreType.DMA,
        ],
    )
    def k(msg_hbm, dst_hbm, z_hbm, out_hbm, idx_v, rows_v, acc_sh, sem):
        cid = lax.axis_index("c")
        sid = lax.axis_index("s")
        base = cid * epc + sid * eps
        rowbase = cid * (epc // _IDX_CHUNK) + sid * nch
        pltpu.sync_copy(dst_hbm.at[pl.ds(rowbase, nch)], idx_v)
        pltpu.sync_copy(msg_hbm.at[pl.ds(base, eps)], rows_v)
        pltpu.sync_copy(
            z_hbm.at[pl.ds(sid * rows_pc, rows_pc)],
            acc_sh.at[pl.ds(sid * rows_pc, rows_pc)],
        )
        plsc.subcore_barrier()
        for j in range(nch):
            pltpu.sync_copy(
                rows_v.at[pl.ds(j * _IDX_CHUNK, _IDX_CHUNK)],
                acc_sh.at[idx_v.at[j]],
                add=True,
            )
        plsc.subcore_barrier()
        pltpu.sync_copy(
            acc_sh.at[pl.ds(sid * rows_pc, rows_pc)],
            out_hbm.at[cid].at[pl.ds(sid * rows_pc, rows_pc)],
        )

    return k(msg, dst2d, zrows)


def _final_body(agg_ref, x_ref, root_ref, bias_ref, g1_ref, be1_ref, g2_ref,
                be2_ref, wlinT_ref, blin_ref, out_ref):
    o = (
        agg_ref[0] + agg_ref[1]
        + jnp.dot(x_ref[...], root_ref[...], preferred_element_type=jnp.float32)
        + bias_ref[...]
    )
    mu = jnp.mean(o, axis=1, keepdims=True)
    var = jnp.mean((o - mu) * (o - mu), axis=1, keepdims=True)
    h1 = (o - mu) * lax.rsqrt(var + 1e-5) * g1_ref[...] + be1_ref[...]
    t = (
        h1
        + jnp.dot(h1, wlinT_ref[...], preferred_element_type=jnp.float32)
        + blin_ref[...]
    )
    mu2 = jnp.mean(t, axis=1, keepdims=True)
    var2 = jnp.mean((t - mu2) * (t - mu2), axis=1, keepdims=True)
    out_ref[...] = (t - mu2) * lax.rsqrt(var2 + 1e-5) * g2_ref[...] + be2_ref[...]


def _finalize(agg2, x, root, bias2, g12, be12, g22, be22, wlinT, blin2):
    n, d = x.shape
    bn = 2000
    grid = n // bn
    rspec = pl.BlockSpec((1, d), lambda i: (0, 0))
    return pl.pallas_call(
        _final_body,
        grid=(grid,),
        in_specs=[
            pl.BlockSpec((_NC, bn, d), lambda i: (0, i, 0)),
            pl.BlockSpec((bn, d), lambda i: (i, 0)),
            pl.BlockSpec((d, d), lambda i: (0, 0)),
            rspec, rspec, rspec, rspec, rspec,
            pl.BlockSpec((d, d), lambda i: (0, 0)),
            rspec,
        ],
        out_specs=pl.BlockSpec((bn, d), lambda i: (i, 0)),
        out_shape=jax.ShapeDtypeStruct((n, d), jnp.float32),
    )(agg2, x, root, bias2, g12, be12, g22, be22, wlinT, blin2)


def kernel(x, edge_index, edge_attr, edge_masks, W_nn, b_nn, W_a1, b_a1,
           W_a2, root, bias, g1, be1, g2, be2, W_lin, b_lin):
    n, d = x.shape
    e = edge_index.shape[1]

    src2d = edge_index[0].astype(jnp.int32).reshape(e // _IDX_CHUNK, _IDX_CHUNK)
    dst2d = edge_index[1].astype(jnp.int32).reshape(e // _IDX_CHUNK, _IDX_CHUNK)

    wnnT = W_nn.T
    bnn2 = b_nn.reshape(1, d)
    a1T = W_a1[:, :d].T
    a2T = W_a1[:, d:2 * d].T
    a3T = W_a1[:, 2 * d:].T
    ba12 = b_a1.reshape(1, d)
    wa2row = W_a2.reshape(1, d)
    wlinT = W_lin.T
    n_pad = ((n + 2047) // 2048) * 2048

    u, zrows = _node_transform(x, wnnT, bnn2, n_pad)
    wj2, wi2 = _sc_gather(u, src2d, dst2d, e, d)
    msg = _edge_mask_messages(wi2, wj2, edge_attr, edge_masks.view(jnp.int8),
                              wnnT, bnn2, a1T, a2T, a3T, ba12, wa2row)
    agg2 = _sc_scatter(msg, dst2d, zrows, n_pad, d, e)
    return _finalize(agg2, x, root, bias.reshape(1, d), g1.reshape(1, d),
                     be1.reshape(1, d), g2.reshape(1, d), be2.reshape(1, d),
                     wlinT, b_lin.reshape(1, d))
